# Initial kernel scaffold; baseline (speedup 1.0000x reference)
import jax
import jax.numpy as jnp
from jax.experimental import pallas as pl


def kernel(x, edge_index, mlp_W, mlp_b, mlp_bn_g, mlp_bn_b, Wg0, Wcon0, Wdis0, Wself0, bself0, gW0, gb0, bn0_g, bn0_b, Wg1, Wcon1, Wdis1, Wself1, bself1, gW1, gb1, cls_W, cls_b):
    raise NotImplementedError("write your pallas kernel here")


# speedup vs baseline: 4.4920x; 4.4920x over previous
import functools

import numpy as np
import jax
import jax.numpy as jnp
from jax import lax
from jax.experimental import pallas as pl
from jax.experimental.pallas import tpu as pltpu
from jax.experimental.pallas import tpu_sc as plsc

f32 = jnp.float32
i32 = jnp.int32

N = 10000
D = 128
H = 128
C = 64
E = 320000
EPR = E + N
TAU = 1.0
EPS = 1e-5
E1 = float(np.e)

NC = 2
NS = 16
L = 16
NW = NC * NS

CHUNK = 128
CH_A = 82
EP = NW * CH_A * CHUNK
NCHUNKS = EP // CHUNK
CH_B = NCHUNKS // NS
NPAD = 10240
NSL = NPAD // NS

BN = 1000
GRID = N // BN


def _full_spec(shape):
    return pl.BlockSpec(shape, lambda i: tuple(0 for _ in shape))


_ROW_SPEC = pl.BlockSpec((BN, H), lambda i: (i, 0))


def _tc_pre_body(x_ref, mw_ref, mb_ref, sg_ref, sb_ref, wc_ref, bc_ref,
                 h_ref, xg_ref, xc_ref, xd_ref, xs_ref):
    xb = x_ref[...]
    hb = jnp.dot(xb, mw_ref[...], preferred_element_type=f32) + mb_ref[...]
    hb = hb * sg_ref[...] + sb_ref[...]
    hb = jnp.maximum(hb, 0.0)
    h_ref[...] = hb
    pj = jnp.dot(hb, wc_ref[...], preferred_element_type=f32) + bc_ref[...]
    xg_ref[...] = pj[:, 0:H]
    xc_ref[...] = pj[:, H:2 * H]
    xd_ref[...] = pj[:, 2 * H:3 * H]
    xs_ref[...] = pj[:, 3 * H:4 * H]


def _tc_pre(x, mw, mb, sg, sb, wc, bc):
    return pl.pallas_call(
        _tc_pre_body,
        grid=(GRID,),
        in_specs=[_ROW_SPEC, _full_spec((H, H)), _full_spec((1, H)),
                  _full_spec((1, H)), _full_spec((1, H)),
                  _full_spec((H, 4 * H)), _full_spec((1, 4 * H))],
        out_specs=[_ROW_SPEC] * 5,
        out_shape=[jax.ShapeDtypeStruct((N, H), f32)] * 5,
    )(x, mw, mb, sg, sb, wc, bc)


def _gate_combine(oc, od, osf, g0, g1, g2, gb):
    l0 = jnp.sum(oc * g0[0:1, 0:H] + od * g0[0:1, H:2 * H]
                 + osf * g0[0:1, 2 * H:3 * H], axis=1, keepdims=True) + gb[0, 0]
    l1 = jnp.sum(oc * g1[0:1, 0:H] + od * g1[0:1, H:2 * H]
                 + osf * g1[0:1, 2 * H:3 * H], axis=1, keepdims=True) + gb[0, 1]
    l2 = jnp.sum(oc * g2[0:1, 0:H] + od * g2[0:1, H:2 * H]
                 + osf * g2[0:1, 2 * H:3 * H], axis=1, keepdims=True) + gb[0, 2]
    m = jnp.maximum(jnp.maximum(l0, l1), l2)
    e0 = jnp.exp(l0 - m)
    e1 = jnp.exp(l1 - m)
    e2 = jnp.exp(l2 - m)
    den = e0 + e1 + e2
    return (e0 * oc + e1 * od + e2 * osf) / den


def _tc_mid_body(h_ref, oc_ref, od_ref, os_ref, g0_ref, g1_ref, g2_ref,
                 gb_ref, sg_ref, sb_ref, wc_ref, bc_ref,
                 ho_ref, xg_ref, xc_ref, xd_ref, xs_ref):
    comb = _gate_combine(oc_ref[...], od_ref[...], os_ref[...],
                         g0_ref[...], g1_ref[...], g2_ref[...], gb_ref[...])
    hn = comb * sg_ref[...] + sb_ref[...]
    hn = jnp.maximum(hn, 0.0) + h_ref[...]
    ho_ref[...] = hn
    pj = jnp.dot(hn, wc_ref[...], preferred_element_type=f32) + bc_ref[...]
    xg_ref[...] = pj[:, 0:H]
    xc_ref[...] = pj[:, H:2 * H]
    xd_ref[...] = pj[:, 2 * H:3 * H]
    xs_ref[...] = pj[:, 3 * H:4 * H]


def _tc_mid(h, oc, od, osf, g0, g1, g2, gb, sg, sb, wc, bc):
    return pl.pallas_call(
        _tc_mid_body,
        grid=(GRID,),
        in_specs=[_ROW_SPEC] * 4
                 + [_full_spec((1, 3 * H))] * 3
                 + [_full_spec((1, H))] * 3
                 + [_full_spec((H, 4 * H)), _full_spec((1, 4 * H))],
        out_specs=[_ROW_SPEC] * 5,
        out_shape=[jax.ShapeDtypeStruct((N, H), f32)] * 5,
    )(h, oc, od, osf, g0, g1, g2, gb, sg, sb, wc, bc)


def _tc_post_body(h_ref, oc_ref, od_ref, os_ref, g0_ref, g1_ref, g2_ref,
                  gb_ref, cw_ref, cb_ref, out_ref):
    comb = _gate_combine(oc_ref[...], od_ref[...], os_ref[...],
                         g0_ref[...], g1_ref[...], g2_ref[...], gb_ref[...])
    hn = comb + h_ref[...]
    out_ref[...] = jnp.dot(hn, cw_ref[...], preferred_element_type=f32) \
        + cb_ref[...]


def _tc_dsum_body(dp_ref, out_ref):
    out_ref[...] = dp_ref[0] + dp_ref[1]


def _tc_dsum(dpart):
    return pl.pallas_call(
        _tc_dsum_body,
        out_shape=jax.ShapeDtypeStruct((2, NPAD), f32),
    )(dpart)


def _tc_post(h, oc, od, osf, g0, g1, g2, gb, cw, cb):
    return pl.pallas_call(
        _tc_post_body,
        grid=(GRID,),
        in_specs=[_ROW_SPEC] * 4
                 + [_full_spec((1, 3 * H))] * 3
                 + [_full_spec((1, H)), _full_spec((H, 128)),
                    _full_spec((1, 128))],
        out_specs=pl.BlockSpec((BN, 128), lambda i: (i, 0)),
        out_shape=jax.ShapeDtypeStruct((N, 128), f32),
    )(h, oc, od, osf, g0, g1, g2, gb, cw, cb)


def _sc_score_body(xg_hbm, row_hbm, col_hbm, expc_hbm, dpart_hbm,
                   row_v, col_v, arows, brows, dcon, ddis, exps_st,
                   red_a, red_b, dsh, sem):
    c = lax.axis_index("c")
    s = lax.axis_index("s")
    wid = c * NS + s
    zero16 = jnp.zeros((L,), f32)
    iota = lax.iota(i32, L)

    def _zero_d(i, carry):
        dcon[pl.ds(i * L, L)] = zero16
        ddis[pl.ds(i * L, L)] = zero16
        return carry
    lax.fori_loop(0, NPAD // L, _zero_d, 0)

    pltpu.sync_copy(row_hbm.at[pl.ds(wid * CH_A, CH_A)], row_v)
    pltpu.sync_copy(col_hbm.at[pl.ds(wid * CH_A, CH_A)], col_v)

    def _chunk(ch, carry):
        cp1 = pltpu.async_copy(xg_hbm.at[row_v.at[ch]], arows, sem)
        cp2 = pltpu.async_copy(xg_hbm.at[col_v.at[ch]], brows, sem)
        cp1.wait()
        cp2.wait()

        def _group(g, gcarry):
            eg = g * L + iota
            chv = jnp.full((L,), ch, i32)
            acc = jnp.zeros((L,), f32)
            for dd in range(H):
                dv = jnp.full((L,), dd, i32)
                av = plsc.load_gather(arows, [eg, dv])
                bv = plsc.load_gather(brows, [eg, dv])
                t = av - bv
                acc = acc + t * t
            ss = acc + 1e-12
            yi = plsc.bitcast(ss, i32)
            yi = 0x5F3759DF - lax.shift_right_logical(yi, 1)
            y = plsc.bitcast(yi, f32)
            for _ in range(3):
                y = y * (1.5 - 0.5 * ss * y * y)
            gv = ss * y
            sgm = 1.0 / (1.0 + jnp.exp(gv * (1.0 / TAU)))
            ec = jnp.exp(sgm)
            ed = E1 / ec
            ids = wid * (CH_A * CHUNK) + ch * CHUNK + g * L + iota
            valid = ids < EPR
            row16 = plsc.load_gather(row_v, [chv, eg])
            plsc.addupdate_scatter(dcon, [row16], ec, mask=valid)
            plsc.addupdate_scatter(ddis, [row16], ed, mask=valid)
            exps_st[pl.ds(g * L, L)] = ec
            return gcarry
        lax.fori_loop(0, CHUNK // L, _group, 0)
        pltpu.sync_copy(exps_st, expc_hbm.at[wid * CH_A + ch])
        return carry
    lax.fori_loop(0, CH_A, _chunk, 0)

    pltpu.sync_copy(dcon, dsh.at[s, 0])
    pltpu.sync_copy(ddis, dsh.at[s, 1])
    plsc.subcore_barrier()
    for b in range(2):
        def _zr(i, carry):
            red_a[pl.ds(i * L, L)] = zero16
            return carry
        lax.fori_loop(0, NSL // L, _zr, 0)

        def _src(j, carry):
            pltpu.sync_copy(dsh.at[j, b, pl.ds(s * NSL, NSL)], red_b)

            def _addv(i, icarry):
                slc = pl.ds(i * L, L)
                red_a[slc] = red_a[slc] + red_b[slc]
                return icarry
            lax.fori_loop(0, NSL // L, _addv, 0)
            return carry
        lax.fori_loop(0, NS, _src, 0)
        pltpu.sync_copy(red_a, dpart_hbm.at[c, b, pl.ds(s * NSL, NSL)])


def _sc_score(xg, row2d, col2d):
    mesh = plsc.VectorSubcoreMesh(core_axis_name="c", subcore_axis_name="s",
                                  num_cores=NC, num_subcores=NS)
    fn = pl.kernel(
        _sc_score_body,
        out_type=[jax.ShapeDtypeStruct((NCHUNKS, CHUNK), f32),
                  jax.ShapeDtypeStruct((NC, 2, NPAD), f32)],
        mesh=mesh,
        compiler_params=pltpu.CompilerParams(use_tc_tiling_on_sc=False, needs_layout_passes=False),
        scratch_types=[
            pltpu.VMEM((CH_A, CHUNK), i32),
            pltpu.VMEM((CH_A, CHUNK), i32),
            pltpu.VMEM((CHUNK, H), f32),
            pltpu.VMEM((CHUNK, H), f32),
            pltpu.VMEM((NPAD,), f32),
            pltpu.VMEM((NPAD,), f32),
            pltpu.VMEM((CHUNK,), f32),
            pltpu.VMEM((NSL,), f32),
            pltpu.VMEM((NSL,), f32),
            pltpu.VMEM_SHARED((NS, 2, NPAD), f32),
            pltpu.SemaphoreType.DMA,
        ],
    )
    return fn(xg, row2d, col2d)


SB = 4
NSB = CH_B // SB


def _sc_agg_body(xcat_hbm, row_hbm, col_hbm, expc_hbm, dtot_hbm, out_hbm,
                 rowb, colb, expb, dloc, xrows, wbuf, acc_sh, sem):
    c = lax.axis_index("c")
    s = lax.axis_index("s")
    zero16 = jnp.zeros((L,), f32)
    iota = lax.iota(i32, L)
    coff = c * N

    pltpu.sync_copy(dtot_hbm.at[c], dloc)

    def _zrow(r, carry):
        for j in range(H // L):
            xrows[r, pl.ds(j * L, L)] = zero16
        return carry
    lax.fori_loop(0, CHUNK, _zrow, 0)
    for i in range(NSL // CHUNK):
        pltpu.sync_copy(xrows, acc_sh.at[pl.ds(s * NSL + i * CHUNK, CHUNK)])
    plsc.subcore_barrier()

    def _sblock(u, carry):
        base = s * CH_B + u * SB
        pltpu.sync_copy(row_hbm.at[pl.ds(base, SB)], rowb)
        pltpu.sync_copy(col_hbm.at[pl.ds(base, SB)], colb)
        pltpu.sync_copy(expc_hbm.at[pl.ds(base, SB)], expb)

        def _shift(k, kcarry):
            slc = pl.ds(k * L, L)
            for q in range(SB):
                rowb[q, slc] = rowb[q, slc] + coff
            return kcarry
        lax.fori_loop(0, CHUNK // L, _shift, 0)

        for q in range(SB):
            pltpu.async_copy(xcat_hbm.at[rowb.at[q]], xrows, sem).wait()
            qv = jnp.full((L,), q, i32)

            def _group(g, gcarry):
                eg = g * L + iota
                ecv = plsc.load_gather(expb, [qv, eg])
                rsh = plsc.load_gather(rowb, [qv, eg])
                rd = rsh - coff
                dv = plsc.load_gather(dloc, [rd])
                cvec = jnp.full((L,), c, i32)
                num = jnp.where(cvec == 0, ecv, E1 / ecv)
                w = num / dv
                ids = (base + q) * CHUNK + g * L + iota
                w = jnp.where(ids < EPR, w, 0.0)
                wbuf[pl.ds(g * L, L)] = w
                return gcarry
            lax.fori_loop(0, CHUNK // L, _group, 0)

            def _scale(e, ecarry):
                ws = plsc.load_gather(wbuf, [jnp.full((L,), e, i32)])
                for j in range(H // L):
                    slc = pl.ds(j * L, L)
                    xrows[e, slc] = xrows[e, slc] * ws
                return ecarry
            lax.fori_loop(0, CHUNK, _scale, 0)

            pltpu.sync_copy(xrows, acc_sh.at[colb.at[q]], add=True)
        return carry
    lax.fori_loop(0, NSB, _sblock, 0)

    plsc.subcore_barrier()
    for i in range(NSL // CHUNK):
        pltpu.sync_copy(acc_sh.at[pl.ds(s * NSL + i * CHUNK, CHUNK)], xrows)
        pltpu.sync_copy(xrows, out_hbm.at[c, pl.ds(s * NSL + i * CHUNK, CHUNK)])


def _sc_agg(xcat, row2d, col2d, expc, dtot):
    mesh = plsc.VectorSubcoreMesh(core_axis_name="c", subcore_axis_name="s",
                                  num_cores=NC, num_subcores=NS)
    fn = pl.kernel(
        _sc_agg_body,
        out_type=jax.ShapeDtypeStruct((NC, NPAD, H), f32),
        mesh=mesh,
        compiler_params=pltpu.CompilerParams(use_tc_tiling_on_sc=False, needs_layout_passes=False),
        scratch_types=[
            pltpu.VMEM((SB, CHUNK), i32),
            pltpu.VMEM((SB, CHUNK), i32),
            pltpu.VMEM((SB, CHUNK), f32),
            pltpu.VMEM((NPAD,), f32),
            pltpu.VMEM((CHUNK, H), f32),
            pltpu.VMEM((CHUNK,), f32),
            pltpu.VMEM_SHARED((NPAD, H), f32),
            pltpu.SemaphoreType.DMA,
        ],
    )
    return fn(xcat, row2d, col2d, expc, dtot)


def kernel(x, edge_index, mlp_W, mlp_b, mlp_bn_g, mlp_bn_b,
           Wg0, Wcon0, Wdis0, Wself0, bself0, gW0, gb0,
           bn0_g, bn0_b,
           Wg1, Wcon1, Wdis1, Wself1, bself1, gW1, gb1,
           cls_W, cls_b):
    sl = jnp.arange(N, dtype=edge_index.dtype)
    row = jnp.concatenate([edge_index[0], sl]).astype(i32)
    col = jnp.concatenate([edge_index[1], sl]).astype(i32)
    row2d = jnp.pad(row, (0, EP - EPR)).reshape(NCHUNKS, CHUNK)
    col2d = jnp.pad(col, (0, EP - EPR)).reshape(NCHUNKS, CHUNK)

    inv = jnp.float32(1.0 / np.sqrt(1.0 + EPS))
    msg = (mlp_bn_g * inv).reshape(1, H)
    msb = mlp_bn_b.reshape(1, H)
    sg0 = (bn0_g * inv).reshape(1, H)
    sb0 = bn0_b.reshape(1, H)
    wc0 = jnp.concatenate([Wg0.T, Wcon0.T, Wdis0.T, Wself0.T], axis=1)
    bc0 = jnp.concatenate([jnp.zeros((3 * H,), f32), bself0]).reshape(1, 4 * H)
    wc1 = jnp.concatenate([Wg1.T, Wcon1.T, Wdis1.T, Wself1.T], axis=1)
    bc1 = jnp.concatenate([jnp.zeros((3 * H,), f32), bself1]).reshape(1, 4 * H)

    h0, xg0, xc0, xd0, xs0 = _tc_pre(x, mlp_W.T, mlp_b.reshape(1, H),
                                     msg, msb, wc0, bc0)

    expc0, dpart0 = _sc_score(xg0, row2d, col2d)
    dtot0 = _tc_dsum(dpart0)
    xcat0 = jnp.concatenate([xc0, xd0], axis=0)
    outb0 = _sc_agg(xcat0, row2d, col2d, expc0, dtot0)
    oc0 = outb0[0, :N, :]
    od0 = outb0[1, :N, :]

    gbp0 = jnp.zeros((1, H), f32).at[0, :3].set(gb0)
    h1, xg1, xc1, xd1, xs1 = _tc_mid(h0, oc0, od0, xs0,
                                     gW0[0:1], gW0[1:2], gW0[2:3], gbp0,
                                     sg0, sb0, wc1, bc1)

    expc1, dpart1 = _sc_score(xg1, row2d, col2d)
    dtot1 = _tc_dsum(dpart1)
    xcat1 = jnp.concatenate([xc1, xd1], axis=0)
    outb1 = _sc_agg(xcat1, row2d, col2d, expc1, dtot1)
    oc1 = outb1[0, :N, :]
    od1 = outb1[1, :N, :]

    gbp1 = jnp.zeros((1, H), f32).at[0, :3].set(gb1)
    cwp = jnp.zeros((H, 128), f32).at[:, :C].set(cls_W.T)
    cbp = jnp.zeros((1, 128), f32).at[0, :C].set(cls_b)
    logits = _tc_post(h1, oc1, od1, xs1,
                      gW1[0:1], gW1[1:2], gW1[2:3], gbp1, cwp, cbp)
    return logits[:, :C]
